# baseline (speedup 1.0000x reference)
import functools

import jax
import jax.numpy as jnp
from jax import lax
from jax.experimental import pallas as pl
from jax.experimental.pallas import tpu as pltpu
from jax.experimental.pallas import tpu_sc as plsc

B, S = 4, 2048
E = 8
D = 1024
EXP_D = 128
C = 8
GATE_H = 256
CP = 16
NC, NS, L = 2, 16, 16

SAMPLES_PER_SC = B // NC
A_CHUNKS = NS // SAMPLES_PER_SC
A_IDX = S // A_CHUNKS
A_ROWS = 16
A_GATHERS = A_IDX // A_ROWS
DSL = D // NS

B_QUARTERS = 4
B_IDX = S // B_QUARTERS
B_ROWS = 64
B_GATHERS = B_IDX // B_ROWS

_I16 = lambda: lax.iota(jnp.int32, L)


def _splat(vec, lane):
    dnums = lax.GatherDimensionNumbers(
        offset_dims=(), collapsed_slice_dims=(0,), start_index_map=(0,))
    idx = jnp.full((L, 1), lane, jnp.int32)
    return lax.gather(vec, idx, dnums, (1,),
                      mode=lax.GatherScatterMode.PROMISE_IN_BOUNDS)


def _perm_xor(vec, k):
    dnums = lax.GatherDimensionNumbers(
        offset_dims=(), collapsed_slice_dims=(0,), start_index_map=(0,))
    idx = jnp.bitwise_xor(_I16(), k).reshape(L, 1)
    return lax.gather(vec, idx, dnums, (1,),
                      mode=lax.GatherScatterMode.PROMISE_IN_BOUNDS)


def _vmax(v):
    for k in (8, 4, 2, 1):
        v = jnp.maximum(v, _perm_xor(v, k))
    return v


def _vmin(v):
    for k in (8, 4, 2, 1):
        v = jnp.minimum(v, _perm_xor(v, k))
    return v


def _vsum(v):
    for k in (8, 4, 2, 1):
        v = v + _perm_xor(v, k)
    return v


def _accum_rows(buf_v, acc_v, n_rows, n_slices, half_slices, row_unroll):
    zero16 = jnp.zeros((L,), jnp.float32)
    for h0 in range(0, n_slices, half_slices):
        hs = min(half_slices, n_slices - h0)

        def row_body(i, accs, h0=h0, hs=hs):
            out = list(accs)
            for u in range(row_unroll):
                r = i * row_unroll + u
                for j in range(hs):
                    sl = pl.ds((h0 + j) * L, L)
                    out[j] = out[j] + buf_v[r, sl]
            return tuple(out)

        accs = lax.fori_loop(0, n_rows // row_unroll, row_body,
                             (zero16,) * hs)
        for j in range(hs):
            plsc.addupdate(acc_v.at[pl.ds((h0 + j) * L, L)], accs[j])


def _sc_body(vocab,
             x_hbm, emb_hbm, eemb_hbm, w1_hbm, b1_hbm, w2_hbm, b2_hbm,
             ew_hbm, eb_hbm, term_hbm,
             idxa_v, bufa0_v, bufa1_v, acca_v,
             w1_v, b1_v, w2_v, b2_v,
             reda_v, pooled_v, hpartout_v, hrow_v, h_v, route_v,
             idxb_v, bufb0_v, bufb1_v, accb_v,
             ew_v, eb_v, pered_v, outv_v,
             spool_s, sh_s, sroute_s, spe_s,
             sem0, sem1, semw):
    cid = lax.axis_index("c")
    sid = lax.axis_index("s")
    zero16 = jnp.zeros((L,), jnp.float32)
    sems = (sem0, sem1)

    b_loc = sid // A_CHUNKS
    chunk = sid % A_CHUNKS
    b_glob = SAMPLES_PER_SC * cid + b_loc
    pltpu.sync_copy(x_hbm.at[pl.ds(b_glob * S + chunk * A_IDX, A_IDX)],
                    idxa_v)

    bufs_a = (bufa0_v, bufa1_v)

    def a_start(ci):
        return pltpu.async_copy(
            emb_hbm.at[idxa_v.at[pl.ds(ci * A_ROWS, A_ROWS)]],
            bufs_a[ci % 2], sems[ci % 2])

    d_prev = a_start(0)

    dw1 = pltpu.async_copy(w1_hbm.at[pl.ds(sid * DSL, DSL)], w1_v, semw)
    for j in range(D // L):
        acca_v[pl.ds(j * L, L)] = zero16

    for ci in range(A_GATHERS):
        d_next = a_start(ci + 1) if ci + 1 < A_GATHERS else None
        d_prev.wait()
        _accum_rows(bufs_a[ci % 2], acca_v, A_ROWS, D // L,
                    half_slices=16, row_unroll=1)
        d_prev = d_next

    pltpu.sync_copy(acca_v, spool_s.at[sid])
    dw1.wait()
    with jax.named_scope("pool_barrier"):
        plsc.subcore_barrier()


    for bl in range(SAMPLES_PER_SC):
        for r in range(A_CHUNKS):
            pltpu.sync_copy(
                spool_s.at[bl * A_CHUNKS + r, pl.ds(sid * DSL, DSL)],
                reda_v.at[r])
        for j in range(DSL // L):
            sl = pl.ds(j * L, L)
            t = reda_v[0, sl]
            for r in range(1, A_CHUNKS):
                t = t + reda_v[r, sl]
            pooled_v[bl, pl.ds(j * L, L)] = t * (1.0 / S)
    for bl in range(SAMPLES_PER_SC):
        def dchunk_body(dc, haccs, bl=bl):
            pv = pooled_v[bl, pl.ds(dc * L, L)]
            out = list(haccs)
            for t in range(L):
                s = _splat(pv, t)
                for hc in range(GATE_H // L):
                    out[hc] = out[hc] + s * w1_v[dc * L + t,
                                                 pl.ds(hc * L, L)]
            return tuple(out)

        haccs = lax.fori_loop(0, DSL // L, dchunk_body,
                              (zero16,) * (GATE_H // L))
        for hc in range(GATE_H // L):
            hpartout_v[bl, pl.ds(hc * L, L)] = haccs[hc]
    pltpu.sync_copy(hpartout_v, sh_s.at[sid])

    @pl.when(sid < SAMPLES_PER_SC)
    def _():
        pltpu.sync_copy(w2_hbm, w2_v)
        pltpu.sync_copy(b1_hbm, b1_v)
        pltpu.sync_copy(b2_hbm, b2_v)

    with jax.named_scope("h_barrier"):
        plsc.subcore_barrier()


    @pl.when(sid < SAMPLES_PER_SC)
    def _():
        bl = sid
        haccs = [zero16] * (GATE_H // L)
        for r in range(NS):
            pltpu.sync_copy(sh_s.at[r, bl], hrow_v)
            for hc in range(GATE_H // L):
                haccs[hc] = haccs[hc] + hrow_v[pl.ds(hc * L, L)]
        for hc in range(GATE_H // L):
            sl = pl.ds(hc * L, L)
            h_v[sl] = jnp.maximum(haccs[hc] + b1_v[sl], 0.0)

        def w2_body(jc, g):
            hv = h_v[pl.ds(jc * L, L)]
            g0, g1 = g
            for t in range(0, L, 2):
                g0 = g0 + _splat(hv, t) * w2_v[jc * L + t, :]
                g1 = g1 + _splat(hv, t + 1) * w2_v[jc * L + t + 1, :]
            return (g0, g1)

        g0, g1 = lax.fori_loop(0, GATE_H // L, w2_body, (zero16, zero16))
        iota = _I16()
        g = jnp.where(iota < E, g0 + g1 + b2_v[...], -1e30)
        m = _vmax(g)
        p = jnp.exp(g - m)
        probs = p / _vsum(p)
        t1 = _vmax(probs)
        e1 = _vmin(jnp.where(probs >= t1, iota, L))
        probs2 = jnp.where(iota == e1, -1.0, probs)
        t2 = _vmax(probs2)
        e2 = _vmin(jnp.where(probs2 >= t2, iota, L))
        wsum = t1 + t2
        route = jnp.where(
            iota == 0, e1.astype(jnp.float32),
            jnp.where(iota == 1, e2.astype(jnp.float32),
                      jnp.where(iota == 2, t1 / wsum,
                                jnp.where(iota == 3, t2 / wsum, 0.0))))
        route_v[...] = route
        pltpu.sync_copy(route_v, sroute_s.at[bl])

    with jax.named_scope("route_barrier"):
        plsc.subcore_barrier()


    pair = sid // B_QUARTERS
    q = sid % B_QUARTERS
    pb_loc = pair // 2
    pk = pair % 2
    pb_glob = SAMPLES_PER_SC * cid + pb_loc
    pltpu.sync_copy(sroute_s.at[pb_loc], route_v)
    rv = route_v[...]
    rv_i = rv.astype(jnp.int32)
    e_sel = jnp.where(pk == 0, rv_i[0], rv_i[1])
    w_sel_v = jnp.where(pk == 0, _splat(rv, 2), _splat(rv, 3))

    pltpu.sync_copy(x_hbm.at[pl.ds(pb_glob * S + q * B_IDX, B_IDX)],
                    idxb_v)
    off = e_sel * vocab
    for j in range(B_IDX // L):
        sl = pl.ds(j * L, L)
        idxb_v[sl] = idxb_v[sl] + off
    for j in range(EXP_D // L):
        accb_v[pl.ds(j * L, L)] = zero16

    bufs_b = (bufb0_v, bufb1_v)

    def b_start(ci):
        return pltpu.async_copy(
            eemb_hbm.at[idxb_v.at[pl.ds(ci * B_ROWS, B_ROWS)]],
            bufs_b[ci % 2], sems[ci % 2])

    d_prev = b_start(0)
    dwe = pltpu.async_copy(ew_hbm.at[pl.ds(e_sel * EXP_D, EXP_D)],
                           ew_v, semw)
    pltpu.sync_copy(eb_hbm.at[e_sel], eb_v)
    for ci in range(B_GATHERS):
        d_next = b_start(ci + 1) if ci + 1 < B_GATHERS else None
        d_prev.wait()
        _accum_rows(bufs_b[ci % 2], accb_v, B_ROWS, EXP_D // L,
                    half_slices=8, row_unroll=4)
        d_prev = d_next
    pltpu.sync_copy(accb_v, spe_s.at[sid])
    dwe.wait()
    with jax.named_scope("pe_barrier"):
        plsc.subcore_barrier()

    @pl.when(q == 0)
    def _():
        pltpu.sync_copy(spe_s.at[pl.ds(sid, B_QUARTERS)], pered_v)
        for j in range(EXP_D // L):
            sl = pl.ds(j * L, L)
            accb_v[sl] = (pered_v[0, sl] + pered_v[1, sl]
                          + pered_v[2, sl] + pered_v[3, sl])

        def y_body(dc, y):
            pv = accb_v[pl.ds(dc * L, L)]
            y0, y1 = y
            for t in range(0, L, 2):
                y0 = y0 + _splat(pv, t) * ew_v[dc * L + t, :]
                y1 = y1 + _splat(pv, t + 1) * ew_v[dc * L + t + 1, :]
            return (y0, y1)

        y0, y1 = lax.fori_loop(0, EXP_D // L, y_body, (zero16, zero16))
        outv_v[...] = (y0 + y1) * (w_sel_v * (1.0 / S)) + w_sel_v * eb_v[...]
        pltpu.sync_copy(outv_v,
                        term_hbm.at[SAMPLES_PER_SC * cid + pb_loc, pk])


def _moe_fused(x_flat, emb, eemb_flat, w1, b1, w2p, b2p, ewp_flat, ebp):
    vocab = emb.shape[0]
    mesh = plsc.VectorSubcoreMesh(core_axis_name="c", subcore_axis_name="s")
    body = functools.partial(_sc_body, vocab)
    return pl.kernel(
        body,
        out_type=jax.ShapeDtypeStruct((B, 2, CP), jnp.float32),
        mesh=mesh,
        scratch_types=[
            pltpu.VMEM((A_IDX,), jnp.int32),
            pltpu.VMEM((A_ROWS, D), jnp.float32),
            pltpu.VMEM((A_ROWS, D), jnp.float32),
            pltpu.VMEM((D,), jnp.float32),
            pltpu.VMEM((DSL, GATE_H), jnp.float32),
            pltpu.VMEM((GATE_H,), jnp.float32),
            pltpu.VMEM((GATE_H, CP), jnp.float32),
            pltpu.VMEM((CP,), jnp.float32),
            pltpu.VMEM((A_CHUNKS, DSL), jnp.float32),
            pltpu.VMEM((SAMPLES_PER_SC, DSL), jnp.float32),
            pltpu.VMEM((SAMPLES_PER_SC, GATE_H), jnp.float32),
            pltpu.VMEM((GATE_H,), jnp.float32),
            pltpu.VMEM((GATE_H,), jnp.float32),
            pltpu.VMEM((L,), jnp.float32),
            pltpu.VMEM((B_IDX,), jnp.int32),
            pltpu.VMEM((B_ROWS, EXP_D), jnp.float32),
            pltpu.VMEM((B_ROWS, EXP_D), jnp.float32),
            pltpu.VMEM((EXP_D,), jnp.float32),
            pltpu.VMEM((EXP_D, CP), jnp.float32),
            pltpu.VMEM((CP,), jnp.float32),
            pltpu.VMEM((B_QUARTERS, EXP_D), jnp.float32),
            pltpu.VMEM((CP,), jnp.float32),
            pltpu.VMEM_SHARED((NS, D), jnp.float32),
            pltpu.VMEM_SHARED((NS, SAMPLES_PER_SC, GATE_H),
                              jnp.float32),
            pltpu.VMEM_SHARED((SAMPLES_PER_SC, L), jnp.float32),
            pltpu.VMEM_SHARED((NS, EXP_D), jnp.float32),
            pltpu.SemaphoreType.DMA,
            pltpu.SemaphoreType.DMA,
            pltpu.SemaphoreType.DMA,
        ],
    )(x_flat, emb, eemb_flat, w1, b1, w2p, b2p, ewp_flat, ebp)


def kernel(x, emb, gate_W1, gate_b1, gate_W2, gate_b2, exp_emb, exp_W, exp_b):
    vocab = emb.shape[0]
    x_flat = x.reshape(-1).astype(jnp.int32)
    eemb_flat = exp_emb.reshape(E * vocab, EXP_D)
    w2p = jnp.pad(gate_W2, ((0, 0), (0, CP - E)))
    b2p = jnp.pad(gate_b2, (0, CP - E))
    ewp_flat = jnp.pad(exp_W, ((0, 0), (0, 0), (0, CP - C))).reshape(
        E * EXP_D, CP)
    ebp = jnp.pad(exp_b, ((0, 0), (0, CP - C)))
    terms = _moe_fused(x_flat, emb, eemb_flat, gate_W1, gate_b1,
                       w2p, b2p, ewp_flat, ebp)
    return terms.sum(axis=1)[:, :C]


# speedup vs baseline: 1.0685x; 1.0685x over previous
import functools

import jax
import jax.numpy as jnp
from jax import lax
from jax.experimental import pallas as pl
from jax.experimental.pallas import tpu as pltpu
from jax.experimental.pallas import tpu_sc as plsc

B, S = 4, 2048
E = 8
D = 1024
EXP_D = 128
C = 8
GATE_H = 256
CP = 16
NC, NS, L = 2, 16, 16

SAMPLES_PER_SC = B // NC
A_CHUNKS = NS // SAMPLES_PER_SC
A_IDX = S // A_CHUNKS
A_ROWS = 16
A_GATHERS = A_IDX // A_ROWS
DSL = D // NS

B_QUARTERS = 4
B_IDX = S // B_QUARTERS
B_ROWS = 64
B_GATHERS = B_IDX // B_ROWS

_I16 = lambda: lax.iota(jnp.int32, L)


def _splat(vec, lane):
    dnums = lax.GatherDimensionNumbers(
        offset_dims=(), collapsed_slice_dims=(0,), start_index_map=(0,))
    idx = jnp.full((L, 1), lane, jnp.int32)
    return lax.gather(vec, idx, dnums, (1,),
                      mode=lax.GatherScatterMode.PROMISE_IN_BOUNDS)


def _perm_xor(vec, k):
    dnums = lax.GatherDimensionNumbers(
        offset_dims=(), collapsed_slice_dims=(0,), start_index_map=(0,))
    idx = jnp.bitwise_xor(_I16(), k).reshape(L, 1)
    return lax.gather(vec, idx, dnums, (1,),
                      mode=lax.GatherScatterMode.PROMISE_IN_BOUNDS)


def _vmax(v):
    for k in (8, 4, 2, 1):
        v = jnp.maximum(v, _perm_xor(v, k))
    return v


def _vmin(v):
    for k in (8, 4, 2, 1):
        v = jnp.minimum(v, _perm_xor(v, k))
    return v


def _vsum(v):
    for k in (8, 4, 2, 1):
        v = v + _perm_xor(v, k)
    return v


def _accum_rows(buf_v, acc_v, n_rows, n_slices, half_slices, row_unroll):
    zero16 = jnp.zeros((L,), jnp.float32)
    for h0 in range(0, n_slices, half_slices):
        hs = min(half_slices, n_slices - h0)

        def row_body(i, accs, h0=h0, hs=hs):
            out = list(accs)
            for u in range(row_unroll):
                r = i * row_unroll + u
                for j in range(hs):
                    sl = pl.ds((h0 + j) * L, L)
                    out[j] = out[j] + buf_v[r, sl]
            return tuple(out)

        accs = lax.fori_loop(0, n_rows // row_unroll, row_body,
                             (zero16,) * hs)
        for j in range(hs):
            plsc.addupdate(acc_v.at[pl.ds((h0 + j) * L, L)], accs[j])


def _sc_body(vocab,
             x_hbm, emb_hbm, eemb_hbm, w1_hbm, b1_hbm, w2_hbm, b2_hbm,
             ew_hbm, eb_hbm, term_hbm,
             idxa_v, bufa0_v, bufa1_v, acca_v,
             w1_v, b1_v, w2_v, b2_v,
             reda_v, pooled_v, hpartout_v, hrow_v, h_v, route_v,
             idxb_v, bufb0_v, bufb1_v, accb_v,
             ew_v, eb_v, pered_v, outv_v,
             spool_s, sh_s, sroute_s, spe_s,
             sem0, sem1, semw):
    cid = lax.axis_index("c")
    sid = lax.axis_index("s")
    zero16 = jnp.zeros((L,), jnp.float32)
    sems = (sem0, sem1)
    pair = sid // B_QUARTERS
    q = sid % B_QUARTERS
    pb_loc = pair // 2
    pk = pair % 2
    pb_glob = SAMPLES_PER_SC * cid + pb_loc
    pbx_base = pb_glob * S + q * B_IDX

    b_loc = sid // A_CHUNKS
    chunk = sid % A_CHUNKS
    b_glob = SAMPLES_PER_SC * cid + b_loc
    pltpu.sync_copy(x_hbm.at[pl.ds(b_glob * S + chunk * A_IDX, A_IDX)],
                    idxa_v)

    def a_start(ci, buf, sem):
        pltpu.async_copy(
            emb_hbm.at[idxa_v.at[pl.ds(ci * A_ROWS, A_ROWS)]], buf, sem)

    def a_wait(buf, sem):
        pltpu.make_async_copy(emb_hbm.at[pl.ds(0, A_ROWS)], buf, sem).wait()

    a_start(0, bufa0_v, sem0)

    dw1 = pltpu.async_copy(w1_hbm.at[pl.ds(sid * DSL, DSL)], w1_v, semw)
    for j in range(D // L):
        acca_v[pl.ds(j * L, L)] = zero16

    def a_body(i, carry):
        a_start(2 * i + 1, bufa1_v, sem1)
        a_wait(bufa0_v, sem0)
        _accum_rows(bufa0_v, acca_v, A_ROWS, D // L,
                    half_slices=16, row_unroll=2)

        @pl.when(i < A_GATHERS // 2 - 1)
        def _():
            a_start(2 * i + 2, bufa0_v, sem0)

        a_wait(bufa1_v, sem1)
        _accum_rows(bufa1_v, acca_v, A_ROWS, D // L,
                    half_slices=16, row_unroll=2)
        return carry

    lax.fori_loop(0, A_GATHERS // 2, a_body, 0)

    pltpu.sync_copy(acca_v, spool_s.at[sid])
    pltpu.sync_copy(x_hbm.at[pl.ds(pbx_base, B_IDX)], idxb_v)
    dw1.wait()
    with jax.named_scope("pool_barrier"):
        plsc.subcore_barrier()


    for bl in range(SAMPLES_PER_SC):
        for r in range(A_CHUNKS):
            pltpu.sync_copy(
                spool_s.at[bl * A_CHUNKS + r, pl.ds(sid * DSL, DSL)],
                reda_v.at[r])
        for j in range(DSL // L):
            sl = pl.ds(j * L, L)
            t = reda_v[0, sl]
            for r in range(1, A_CHUNKS):
                t = t + reda_v[r, sl]
            pooled_v[bl, pl.ds(j * L, L)] = t * (1.0 / S)
    for bl in range(SAMPLES_PER_SC):
        def dchunk_body(dc, haccs, bl=bl):
            pv = pooled_v[bl, pl.ds(dc * L, L)]
            out = list(haccs)
            for t in range(L):
                s = _splat(pv, t)
                for hc in range(GATE_H // L):
                    out[hc] = out[hc] + s * w1_v[dc * L + t,
                                                 pl.ds(hc * L, L)]
            return tuple(out)

        haccs = lax.fori_loop(0, DSL // L, dchunk_body,
                              (zero16,) * (GATE_H // L))
        for hc in range(GATE_H // L):
            hpartout_v[bl, pl.ds(hc * L, L)] = haccs[hc]
    pltpu.sync_copy(hpartout_v, sh_s.at[sid])

    @pl.when(sid < SAMPLES_PER_SC)
    def _():
        pltpu.sync_copy(w2_hbm, w2_v)
        pltpu.sync_copy(b1_hbm, b1_v)
        pltpu.sync_copy(b2_hbm, b2_v)

    with jax.named_scope("h_barrier"):
        plsc.subcore_barrier()


    @pl.when(sid < SAMPLES_PER_SC)
    def _():
        bl = sid
        haccs = [zero16] * (GATE_H // L)
        for r in range(NS):
            pltpu.sync_copy(sh_s.at[r, bl], hrow_v)
            for hc in range(GATE_H // L):
                haccs[hc] = haccs[hc] + hrow_v[pl.ds(hc * L, L)]
        for hc in range(GATE_H // L):
            sl = pl.ds(hc * L, L)
            h_v[sl] = jnp.maximum(haccs[hc] + b1_v[sl], 0.0)

        def w2_body(jc, g):
            hv = h_v[pl.ds(jc * L, L)]
            g0, g1 = g
            for t in range(0, L, 2):
                g0 = g0 + _splat(hv, t) * w2_v[jc * L + t, :]
                g1 = g1 + _splat(hv, t + 1) * w2_v[jc * L + t + 1, :]
            return (g0, g1)

        g0, g1 = lax.fori_loop(0, GATE_H // L, w2_body, (zero16, zero16))
        iota = _I16()
        g = jnp.where(iota < E, g0 + g1 + b2_v[...], -1e30)
        m = _vmax(g)
        p = jnp.exp(g - m)
        probs = p / _vsum(p)
        t1 = _vmax(probs)
        e1 = _vmin(jnp.where(probs >= t1, iota, L))
        probs2 = jnp.where(iota == e1, -1.0, probs)
        t2 = _vmax(probs2)
        e2 = _vmin(jnp.where(probs2 >= t2, iota, L))
        wsum = t1 + t2
        route = jnp.where(
            iota == 0, e1.astype(jnp.float32),
            jnp.where(iota == 1, e2.astype(jnp.float32),
                      jnp.where(iota == 2, t1 / wsum,
                                jnp.where(iota == 3, t2 / wsum, 0.0))))
        route_v[...] = route
        pltpu.sync_copy(route_v, sroute_s.at[bl])

    with jax.named_scope("route_barrier"):
        plsc.subcore_barrier()


    pltpu.sync_copy(sroute_s.at[pb_loc], route_v)
    rv = route_v[...]
    rv_i = rv.astype(jnp.int32)
    e_sel = jnp.where(pk == 0, rv_i[0], rv_i[1])
    w_sel_v = jnp.where(pk == 0, _splat(rv, 2), _splat(rv, 3))

    off = e_sel * vocab
    for j in range(B_IDX // L):
        sl = pl.ds(j * L, L)
        idxb_v[sl] = idxb_v[sl] + off
    for j in range(EXP_D // L):
        accb_v[pl.ds(j * L, L)] = zero16

    bufs_b = (bufb0_v, bufb1_v)

    def b_start(ci):
        return pltpu.async_copy(
            eemb_hbm.at[idxb_v.at[pl.ds(ci * B_ROWS, B_ROWS)]],
            bufs_b[ci % 2], sems[ci % 2])

    d_prev = b_start(0)
    dwe = pltpu.async_copy(ew_hbm.at[pl.ds(e_sel * EXP_D, EXP_D)],
                           ew_v, semw)
    pltpu.sync_copy(eb_hbm.at[e_sel], eb_v)
    for ci in range(B_GATHERS):
        d_next = b_start(ci + 1) if ci + 1 < B_GATHERS else None
        d_prev.wait()
        _accum_rows(bufs_b[ci % 2], accb_v, B_ROWS, EXP_D // L,
                    half_slices=8, row_unroll=4)
        d_prev = d_next
    pltpu.sync_copy(accb_v, spe_s.at[sid])
    dwe.wait()
    with jax.named_scope("pe_barrier"):
        plsc.subcore_barrier()

    @pl.when(q == 0)
    def _():
        pltpu.sync_copy(spe_s.at[pl.ds(sid, B_QUARTERS)], pered_v)
        for j in range(EXP_D // L):
            sl = pl.ds(j * L, L)
            accb_v[sl] = (pered_v[0, sl] + pered_v[1, sl]
                          + pered_v[2, sl] + pered_v[3, sl])

        def y_body(dc, y):
            pv = accb_v[pl.ds(dc * L, L)]
            y0, y1 = y
            for t in range(0, L, 2):
                y0 = y0 + _splat(pv, t) * ew_v[dc * L + t, :]
                y1 = y1 + _splat(pv, t + 1) * ew_v[dc * L + t + 1, :]
            return (y0, y1)

        y0, y1 = lax.fori_loop(0, EXP_D // L, y_body, (zero16, zero16))
        outv_v[...] = (y0 + y1) * (w_sel_v * (1.0 / S)) + w_sel_v * eb_v[...]
        pltpu.sync_copy(outv_v,
                        term_hbm.at[SAMPLES_PER_SC * cid + pb_loc, pk])


def _moe_fused(x_flat, emb, eemb_flat, w1, b1, w2p, b2p, ewp_flat, ebp):
    vocab = emb.shape[0]
    mesh = plsc.VectorSubcoreMesh(core_axis_name="c", subcore_axis_name="s")
    body = functools.partial(_sc_body, vocab)
    return pl.kernel(
        body,
        out_type=jax.ShapeDtypeStruct((B, 2, CP), jnp.float32),
        mesh=mesh,
        scratch_types=[
            pltpu.VMEM((A_IDX,), jnp.int32),
            pltpu.VMEM((A_ROWS, D), jnp.float32),
            pltpu.VMEM((A_ROWS, D), jnp.float32),
            pltpu.VMEM((D,), jnp.float32),
            pltpu.VMEM((DSL, GATE_H), jnp.float32),
            pltpu.VMEM((GATE_H,), jnp.float32),
            pltpu.VMEM((GATE_H, CP), jnp.float32),
            pltpu.VMEM((CP,), jnp.float32),
            pltpu.VMEM((A_CHUNKS, DSL), jnp.float32),
            pltpu.VMEM((SAMPLES_PER_SC, DSL), jnp.float32),
            pltpu.VMEM((SAMPLES_PER_SC, GATE_H), jnp.float32),
            pltpu.VMEM((GATE_H,), jnp.float32),
            pltpu.VMEM((GATE_H,), jnp.float32),
            pltpu.VMEM((L,), jnp.float32),
            pltpu.VMEM((B_IDX,), jnp.int32),
            pltpu.VMEM((B_ROWS, EXP_D), jnp.float32),
            pltpu.VMEM((B_ROWS, EXP_D), jnp.float32),
            pltpu.VMEM((EXP_D,), jnp.float32),
            pltpu.VMEM((EXP_D, CP), jnp.float32),
            pltpu.VMEM((CP,), jnp.float32),
            pltpu.VMEM((B_QUARTERS, EXP_D), jnp.float32),
            pltpu.VMEM((CP,), jnp.float32),
            pltpu.VMEM_SHARED((NS, D), jnp.float32),
            pltpu.VMEM_SHARED((NS, SAMPLES_PER_SC, GATE_H),
                              jnp.float32),
            pltpu.VMEM_SHARED((SAMPLES_PER_SC, L), jnp.float32),
            pltpu.VMEM_SHARED((NS, EXP_D), jnp.float32),
            pltpu.SemaphoreType.DMA,
            pltpu.SemaphoreType.DMA,
            pltpu.SemaphoreType.DMA,
        ],
    )(x_flat, emb, eemb_flat, w1, b1, w2p, b2p, ewp_flat, ebp)


def kernel(x, emb, gate_W1, gate_b1, gate_W2, gate_b2, exp_emb, exp_W, exp_b):
    vocab = emb.shape[0]
    x_flat = x.reshape(-1).astype(jnp.int32)
    eemb_flat = exp_emb.reshape(E * vocab, EXP_D)
    w2p = jnp.pad(gate_W2, ((0, 0), (0, CP - E)))
    b2p = jnp.pad(gate_b2, (0, CP - E))
    ewp_flat = jnp.pad(exp_W, ((0, 0), (0, 0), (0, CP - C))).reshape(
        E * EXP_D, CP)
    ebp = jnp.pad(exp_b, ((0, 0), (0, CP - C)))
    terms = _moe_fused(x_flat, emb, eemb_flat, gate_W1, gate_b1,
                       w2p, b2p, ewp_flat, ebp)
    return terms.sum(axis=1)[:, :C]


# speedup vs baseline: 1.1174x; 1.0457x over previous
import functools

import jax
import jax.numpy as jnp
from jax import lax
from jax.experimental import pallas as pl
from jax.experimental.pallas import tpu as pltpu
from jax.experimental.pallas import tpu_sc as plsc

B, S = 4, 2048
E = 8
D = 1024
EXP_D = 128
C = 8
NC, NS, L = 2, 16, 16
NW = NC * NS

A_CHUNKS = 8
A_IDX = S // A_CHUNKS
A_ROWS = 32
A_GATHERS = A_IDX // A_ROWS

B_ROWS = 128
B_GATHERS = S // B_ROWS


def _accum_rows(buf_v, acc_v, n_rows, n_slices, half_slices, row_unroll):
    zero16 = jnp.zeros((L,), jnp.float32)
    for h0 in range(0, n_slices, half_slices):
        hs = min(half_slices, n_slices - h0)

        def row_body(i, accs, h0=h0, hs=hs):
            out = list(accs)
            for u in range(row_unroll):
                r = i * row_unroll + u
                for j in range(hs):
                    sl = pl.ds((h0 + j) * L, L)
                    out[j] = out[j] + buf_v[r, sl]
            return tuple(out)

        accs = lax.fori_loop(0, n_rows // row_unroll, row_body,
                             (zero16,) * hs)
        for j in range(hs):
            plsc.addupdate(acc_v.at[pl.ds((h0 + j) * L, L)], accs[j])


def _sc_body(vocab, x_hbm, emb_hbm, eemb_hbm, pa_hbm, pb_hbm,
             idxa_v, bufa0_v, bufa1_v, acca_v,
             idxb_v, bufb0_v, bufb1_v, accb_v, sem0, sem1):
    wid = lax.axis_index("s") * NC + lax.axis_index("c")
    zero16 = jnp.zeros((L,), jnp.float32)

    b = wid // A_CHUNKS
    c = wid % A_CHUNKS
    base = b * S + c * A_IDX
    pltpu.sync_copy(x_hbm.at[pl.ds(base, A_IDX)], idxa_v)

    bufs_a = (bufa0_v, bufa1_v)
    sems = (sem0, sem1)

    def a_start(ci):
        return pltpu.async_copy(
            emb_hbm.at[idxa_v.at[pl.ds(ci * A_ROWS, A_ROWS)]],
            bufs_a[ci % 2], sems[ci % 2])

    d_prev = a_start(0)

    e = wid // B
    bb = wid % B
    pltpu.sync_copy(x_hbm.at[pl.ds(bb * S, S)], idxb_v)
    off = e * vocab
    for j in range(S // L):
        sl = pl.ds(j * L, L)
        idxb_v[sl] = idxb_v[sl] + off
    for j in range(D // L):
        acca_v[pl.ds(j * L, L)] = zero16
    for j in range(EXP_D // L):
        accb_v[pl.ds(j * L, L)] = zero16

    for ci in range(A_GATHERS):
        d_next = a_start(ci + 1) if ci + 1 < A_GATHERS else None
        d_prev.wait()
        _accum_rows(bufs_a[ci % 2], acca_v, A_ROWS, D // L,
                    half_slices=32, row_unroll=1)
        d_prev = d_next
    pltpu.sync_copy(acca_v, pa_hbm.at[wid])

    bufs_b = (bufb0_v, bufb1_v)

    def b_start(ci):
        return pltpu.async_copy(
            eemb_hbm.at[idxb_v.at[pl.ds(ci * B_ROWS, B_ROWS)]],
            bufs_b[ci % 2], sems[ci % 2])

    d_prev = b_start(0)
    for ci in range(B_GATHERS):
        d_next = b_start(ci + 1) if ci + 1 < B_GATHERS else None
        d_prev.wait()
        _accum_rows(bufs_b[ci % 2], accb_v, B_ROWS, EXP_D // L,
                    half_slices=8, row_unroll=4)
        d_prev = d_next
    pltpu.sync_copy(accb_v, pb_hbm.at[wid])


def _sc_pools(x_flat, emb, eemb_flat):
    vocab = emb.shape[0]
    mesh = plsc.VectorSubcoreMesh(core_axis_name="c", subcore_axis_name="s")
    body = functools.partial(_sc_body, vocab)
    return pl.kernel(
        body,
        out_type=(
            jax.ShapeDtypeStruct((NW, D), jnp.float32),
            jax.ShapeDtypeStruct((NW, EXP_D), jnp.float32),
        ),
        mesh=mesh,
        scratch_types=[
            pltpu.VMEM((A_IDX,), jnp.int32),
            pltpu.VMEM((A_ROWS, D), jnp.float32),
            pltpu.VMEM((A_ROWS, D), jnp.float32),
            pltpu.VMEM((D,), jnp.float32),
            pltpu.VMEM((S,), jnp.int32),
            pltpu.VMEM((B_ROWS, EXP_D), jnp.float32),
            pltpu.VMEM((B_ROWS, EXP_D), jnp.float32),
            pltpu.VMEM((EXP_D,), jnp.float32),
            pltpu.SemaphoreType.DMA,
            pltpu.SemaphoreType.DMA,
        ],
    )(x_flat, emb, eemb_flat)


def _tc_body(pa_ref, pb_ref, w1_ref, b1_ref, w2_ref, b2_ref,
             expw_ref, expb_ref, out_ref):
    inv_s = 1.0 / S
    pooled = pa_ref[...].reshape(B, A_CHUNKS, D).sum(axis=1) * inv_s
    h = jnp.maximum(pooled @ w1_ref[...] + b1_ref[...], 0.0)
    gates = h @ w2_ref[...] + b2_ref[...]
    m = jnp.max(gates, axis=-1, keepdims=True)
    pexp = jnp.exp(gates - m)
    probs = pexp / jnp.sum(pexp, axis=-1, keepdims=True)
    idx = lax.broadcasted_iota(jnp.int32, (B, E), 1)
    top1 = jnp.max(probs, axis=-1, keepdims=True)
    e1 = jnp.min(jnp.where(probs >= top1, idx, E), axis=-1, keepdims=True)
    m1 = idx == e1
    probs2 = jnp.where(m1, -jnp.inf, probs)
    top2 = jnp.max(probs2, axis=-1, keepdims=True)
    e2 = jnp.min(jnp.where(probs2 >= top2, idx, E), axis=-1, keepdims=True)
    m2 = idx == e2
    denom = top1 + top2
    coeff = (jnp.where(m1, top1, 0.0) + jnp.where(m2, top2, 0.0)) / denom
    ep = pb_ref[...].reshape(E, B, EXP_D) * inv_s
    acc = jnp.zeros((B, C), jnp.float32)
    for e in range(E):
        y = ep[e] @ expw_ref[e] + expb_ref[e]
        acc = acc + coeff[:, e:e + 1] * y
    out_ref[...] = acc


def _tc_combine(pa, pb, gate_W1, gate_b1, gate_W2, gate_b2, exp_W, exp_b):
    return pl.pallas_call(
        _tc_body,
        out_shape=jax.ShapeDtypeStruct((B, C), jnp.float32),
    )(pa, pb, gate_W1, gate_b1, gate_W2, gate_b2, exp_W, exp_b)


def kernel(x, emb, gate_W1, gate_b1, gate_W2, gate_b2, exp_emb, exp_W, exp_b):
    vocab = emb.shape[0]
    x_flat = x.reshape(-1).astype(jnp.int32)
    eemb_flat = exp_emb.reshape(E * vocab, EXP_D)
    pa, pb = _sc_pools(x_flat, emb, eemb_flat)
    return _tc_combine(pa, pb, gate_W1, gate_b1, gate_W2, gate_b2,
                       exp_W, exp_b)
